# Initial kernel scaffold; baseline (speedup 1.0000x reference)
import jax
import jax.numpy as jnp
from jax.experimental import pallas as pl


def kernel(input_data, weight_vec):
    raise NotImplementedError("write your pallas kernel here")


# speedup vs baseline: 7418.3963x; 7418.3963x over previous
import functools

import jax
import jax.numpy as jnp
from jax import lax
from jax.experimental import pallas as pl
from jax.experimental.pallas import tpu as pltpu
from jax.experimental.pallas import tpu_sc as plsc

_GEN = 64
_N = 1048576
_R = 8192
_C = 128
_NUM_ASSOC = _N + 1 - _GEN
_SCALE = float(_NUM_ASSOC - 2)
_MAX_IND = float(_NUM_ASSOC - 1)

_NC = 2
_NS = 16
_NW = _NC * _NS
_B_PER_W = _N // _NW
_K = 4096


def _shift_right_lanes(x, s):
    return jnp.concatenate(
        [jnp.zeros((x.shape[0], s), x.dtype), x[:, : x.shape[1] - s]], axis=1)


def _tc_precompute(x_ref, w_ref, d_ref, ind_ref):
    w = w_ref[...]
    incl = w
    for s in (1, 2, 4, 8, 16, 32, 64):
        incl = incl + _shift_right_lanes(incl, s)
    excl = _shift_right_lanes(incl, 1)
    row_tot = jnp.broadcast_to(incl[:, _C - 1:_C], (_R, _C))
    nxt = jnp.concatenate(
        [incl[1:, :], jnp.zeros((1, _C), incl.dtype)], axis=0)
    nxt_r65 = _shift_right_lanes(nxt, 65)
    incl_l63 = jnp.concatenate(
        [incl[:, 63:], jnp.zeros((_R, 63), incl.dtype)], axis=1)
    c = lax.broadcasted_iota(jnp.int32, (_R, _C), 1)
    d_ref[...] = jnp.where(c <= 64, incl_l63 - excl,
                           row_tot - excl + nxt_r65)

    p = _SCALE * x_ref[...] + 1.0
    p = jnp.where(p < 1.0, 1.0, p)
    p = jnp.where(p > _MAX_IND, _MAX_IND, p)
    ind_ref[...] = jnp.floor(p).astype(jnp.int32)


@functools.cache
def _make_sc_gather():
    mesh = plsc.VectorSubcoreMesh(core_axis_name="c", subcore_axis_name="s")

    @functools.partial(
        pl.kernel,
        out_type=jax.ShapeDtypeStruct((_N,), jnp.float32),
        mesh=mesh,
        scratch_types=[
            pltpu.VMEM((_K,), jnp.int32),
            pltpu.VMEM((_K,), jnp.float32),
            pltpu.SemaphoreType.DMA,
        ],
    )
    def _sc_gather(d_hbm, ind_hbm, out_hbm, idx_v, val_v, sem):
        wid = lax.axis_index("s") * _NC + lax.axis_index("c")
        base = wid * _B_PER_W

        def step(i, carry):
            off = base + i * _K
            pltpu.sync_copy(ind_hbm.at[pl.ds(off, _K)], idx_v)
            pltpu.async_copy(d_hbm.at[idx_v], val_v, sem).wait()
            pltpu.sync_copy(val_v, out_hbm.at[pl.ds(off, _K)])
            return carry

        lax.fori_loop(0, _B_PER_W // _K, step, 0)

    return _sc_gather


def kernel(input_data, weight_vec):
    x2 = input_data.reshape(_R, _C)
    w2 = weight_vec.reshape(_R, _C)
    d2, ind2 = pl.pallas_call(
        _tc_precompute,
        out_shape=(
            jax.ShapeDtypeStruct((_R, _C), jnp.float32),
            jax.ShapeDtypeStruct((_R, _C), jnp.int32),
        ),
    )(x2, w2)
    return _make_sc_gather()(d2.reshape(_N), ind2.reshape(_N))


# speedup vs baseline: 7893.1668x; 1.0640x over previous
import functools

import jax
import jax.numpy as jnp
from jax import lax
from jax.experimental import pallas as pl
from jax.experimental.pallas import tpu as pltpu
from jax.experimental.pallas import tpu_sc as plsc

_GEN = 64
_N = 1048576
_R = 8192
_C = 128
_RB = 512
_G = _R // _RB
_NUM_ASSOC = _N + 1 - _GEN
_SCALE = float(_NUM_ASSOC - 2)
_MAX_IND = float(_NUM_ASSOC - 1)

_NC = 2
_NS = 16
_NW = _NC * _NS
_B_PER_W = _N // _NW
_K = 8192
_NCH = _B_PER_W // _K


def _shift_right_lanes(x, s):
    return jnp.concatenate(
        [jnp.zeros((x.shape[0], s), x.dtype), x[:, : x.shape[1] - s]], axis=1)


def _row_prefix(w):
    incl = w
    for s in (1, 2, 4, 8, 16, 32, 64):
        incl = incl + _shift_right_lanes(incl, s)
    return incl


def _tc_precompute(x_ref, w_ref, wn_ref, d_ref, ind_ref):
    rb = w_ref.shape[0]
    incl = _row_prefix(w_ref[...])
    excl = _shift_right_lanes(incl, 1)
    row_tot = jnp.broadcast_to(incl[:, _C - 1:_C], (rb, _C))
    nxt_row = _row_prefix(wn_ref[0:1, :])
    nxt = jnp.concatenate([incl[1:, :], nxt_row], axis=0)
    nxt_r65 = _shift_right_lanes(nxt, 65)
    incl_l63 = jnp.concatenate(
        [incl[:, 63:], jnp.zeros((rb, 63), incl.dtype)], axis=1)
    c = lax.broadcasted_iota(jnp.int32, (rb, _C), 1)
    d_ref[...] = jnp.where(c <= 64, incl_l63 - excl,
                           row_tot - excl + nxt_r65)

    p = _SCALE * x_ref[...] + 1.0
    p = jnp.where(p < 1.0, 1.0, p)
    p = jnp.where(p > _MAX_IND, _MAX_IND, p)
    ind_ref[...] = jnp.floor(p).astype(jnp.int32)


@functools.cache
def _make_sc_gather():
    mesh = plsc.VectorSubcoreMesh(core_axis_name="c", subcore_axis_name="s")

    @functools.partial(
        pl.kernel,
        out_type=jax.ShapeDtypeStruct((_N,), jnp.float32),
        mesh=mesh,
        scratch_types=[
            pltpu.VMEM((_K,), jnp.int32),
            pltpu.VMEM((_K,), jnp.int32),
            pltpu.VMEM((_K,), jnp.float32),
            pltpu.VMEM((_K,), jnp.float32),
            pltpu.SemaphoreType.DMA((2,)),
            pltpu.SemaphoreType.DMA((2,)),
            pltpu.SemaphoreType.DMA((2,)),
        ],
    )
    def _sc_gather(d_hbm, ind_hbm, out_hbm, idx0, idx1, val0, val1,
                   sem_i, sem_g, sem_o):
        idx_v = [idx0, idx1]
        val_v = [val0, val1]
        wid = lax.axis_index("s") * _NC + lax.axis_index("c")
        base = wid * _B_PER_W

        cp_i = [None, None]
        cp_o = [None, None]
        cp_i[0] = pltpu.async_copy(
            ind_hbm.at[pl.ds(base, _K)], idx_v[0], sem_i.at[0])
        for k in range(_NCH):
            b = k % 2
            if k + 1 < _NCH:
                cp_i[1 - b] = pltpu.async_copy(
                    ind_hbm.at[pl.ds(base + (k + 1) * _K, _K)],
                    idx_v[1 - b], sem_i.at[1 - b])
            cp_i[b].wait()
            if k >= 2:
                cp_o[b].wait()
            pltpu.async_copy(
                d_hbm.at[idx_v[b]], val_v[b], sem_g.at[b]).wait()
            cp_o[b] = pltpu.async_copy(
                val_v[b], out_hbm.at[pl.ds(base + k * _K, _K)], sem_o.at[b])
        for k in range(max(_NCH - 2, 0), _NCH):
            cp_o[k % 2].wait()

    return _sc_gather


def kernel(input_data, weight_vec):
    x2 = input_data.reshape(_R, _C)
    w2 = weight_vec.reshape(_R, _C)
    d2, ind2 = pl.pallas_call(
        _tc_precompute,
        grid=(_G,),
        in_specs=[
            pl.BlockSpec((_RB, _C), lambda i: (i, 0)),
            pl.BlockSpec((_RB, _C), lambda i: (i, 0)),
            pl.BlockSpec((_RB, _C), lambda i: (jnp.minimum(i + 1, _G - 1), 0)),
        ],
        out_specs=(
            pl.BlockSpec((_RB, _C), lambda i: (i, 0)),
            pl.BlockSpec((_RB, _C), lambda i: (i, 0)),
        ),
        out_shape=(
            jax.ShapeDtypeStruct((_R, _C), jnp.float32),
            jax.ShapeDtypeStruct((_R, _C), jnp.int32),
        ),
    )(x2, w2, w2)
    return _make_sc_gather()(d2.reshape(_N), ind2.reshape(_N))


# speedup vs baseline: 8518.7931x; 1.0793x over previous
import functools

import jax
import jax.numpy as jnp
from jax import lax
from jax.experimental import pallas as pl
from jax.experimental.pallas import tpu as pltpu
from jax.experimental.pallas import tpu_sc as plsc

_GEN = 64
_N = 1048576
_R = 8192
_C = 128
_RB = 512
_G = _R // _RB
_NUM_ASSOC = _N + 1 - _GEN
_SCALE = float(_NUM_ASSOC - 2)
_MAX_IND = float(_NUM_ASSOC - 1)

_NC = 2
_NS = 16
_NW = _NC * _NS
_B_PER_W = _N // _NW
_K = 8192
_NCH = _B_PER_W // _K


def _tc_precompute(x_ref, w_ref, wn_ref, d_ref, ind_ref):
    w = w_ref[...]
    wsh = jnp.concatenate([w[1:, :], wn_ref[0:1, :]], axis=0)
    ii = lax.broadcasted_iota(jnp.int32, (_C, _C), 0)
    cc = lax.broadcasted_iota(jnp.int32, (_C, _C), 1)
    a_mask = ((ii >= cc) & (ii <= cc + 63)).astype(jnp.float32)
    b_mask = (ii <= cc - 65).astype(jnp.float32)
    d_ref[...] = (
        jnp.dot(w, a_mask, preferred_element_type=jnp.float32,
                precision=lax.Precision.HIGHEST)
        + jnp.dot(wsh, b_mask, preferred_element_type=jnp.float32,
                  precision=lax.Precision.HIGHEST))

    p = _SCALE * x_ref[...] + 1.0
    p = jnp.where(p < 1.0, 1.0, p)
    p = jnp.where(p > _MAX_IND, _MAX_IND, p)
    ind_ref[...] = jnp.floor(p).astype(jnp.int32)


@functools.cache
def _make_sc_gather():
    mesh = plsc.VectorSubcoreMesh(core_axis_name="c", subcore_axis_name="s")

    @functools.partial(
        pl.kernel,
        out_type=jax.ShapeDtypeStruct((_N,), jnp.float32),
        mesh=mesh,
        scratch_types=[
            pltpu.VMEM((_K,), jnp.int32),
            pltpu.VMEM((_K,), jnp.int32),
            pltpu.VMEM((_K,), jnp.float32),
            pltpu.VMEM((_K,), jnp.float32),
            pltpu.SemaphoreType.DMA((2,)),
            pltpu.SemaphoreType.DMA((2,)),
            pltpu.SemaphoreType.DMA((2,)),
        ],
    )
    def _sc_gather(d_hbm, ind_hbm, out_hbm, idx0, idx1, val0, val1,
                   sem_i, sem_g, sem_o):
        idx_v = [idx0, idx1]
        val_v = [val0, val1]
        wid = lax.axis_index("s") * _NC + lax.axis_index("c")
        base = wid * _B_PER_W

        cp_i = [None, None]
        cp_o = [None, None]
        cp_i[0] = pltpu.async_copy(
            ind_hbm.at[pl.ds(base, _K)], idx_v[0], sem_i.at[0])
        for k in range(_NCH):
            b = k % 2
            if k + 1 < _NCH:
                cp_i[1 - b] = pltpu.async_copy(
                    ind_hbm.at[pl.ds(base + (k + 1) * _K, _K)],
                    idx_v[1 - b], sem_i.at[1 - b])
            cp_i[b].wait()
            if k >= 2:
                cp_o[b].wait()
            pltpu.async_copy(
                d_hbm.at[idx_v[b]], val_v[b], sem_g.at[b]).wait()
            cp_o[b] = pltpu.async_copy(
                val_v[b], out_hbm.at[pl.ds(base + k * _K, _K)], sem_o.at[b])
        for k in range(max(_NCH - 2, 0), _NCH):
            cp_o[k % 2].wait()

    return _sc_gather


def kernel(input_data, weight_vec):
    x2 = input_data.reshape(_R, _C)
    w2 = weight_vec.reshape(_R, _C)
    d2, ind2 = pl.pallas_call(
        _tc_precompute,
        grid=(_G,),
        in_specs=[
            pl.BlockSpec((_RB, _C), lambda i: (i, 0)),
            pl.BlockSpec((_RB, _C), lambda i: (i, 0)),
            pl.BlockSpec(
                (8, _C),
                lambda i: (jnp.minimum((i + 1) * (_RB // 8), _R // 8 - 1), 0)),
        ],
        out_specs=(
            pl.BlockSpec((_RB, _C), lambda i: (i, 0)),
            pl.BlockSpec((_RB, _C), lambda i: (i, 0)),
        ),
        out_shape=(
            jax.ShapeDtypeStruct((_R, _C), jnp.float32),
            jax.ShapeDtypeStruct((_R, _C), jnp.int32),
        ),
    )(x2, w2, w2)
    return _make_sc_gather()(d2.reshape(_N), ind2.reshape(_N))


# speedup vs baseline: 8660.0839x; 1.0166x over previous
import functools

import jax
import jax.numpy as jnp
from jax import lax
from jax.experimental import pallas as pl
from jax.experimental.pallas import tpu as pltpu
from jax.experimental.pallas import tpu_sc as plsc

_GEN = 64
_N = 1048576
_R = 8192
_C = 128
_RB = 512
_G = _R // _RB
_NUM_ASSOC = _N + 1 - _GEN
_SCALE = float(_NUM_ASSOC - 2)
_MAX_IND = float(_NUM_ASSOC - 1)

_NC = 2
_NS = 16
_NW = _NC * _NS
_B_PER_W = _N // _NW
_K = 8192
_NCH = _B_PER_W // _K
_L = 16


def _tc_precompute(w_ref, wn_ref, d_ref):
    w = w_ref[...]
    wsh = jnp.concatenate([w[1:, :], wn_ref[0:1, :]], axis=0)
    ii = lax.broadcasted_iota(jnp.int32, (_C, _C), 0)
    cc = lax.broadcasted_iota(jnp.int32, (_C, _C), 1)
    a_mask = ((ii >= cc) & (ii <= cc + 63)).astype(jnp.float32)
    b_mask = (ii <= cc - 65).astype(jnp.float32)
    d_ref[...] = (
        jnp.dot(w, a_mask, preferred_element_type=jnp.float32,
                precision=lax.Precision.HIGHEST)
        + jnp.dot(wsh, b_mask, preferred_element_type=jnp.float32,
                  precision=lax.Precision.HIGHEST))


@functools.cache
def _make_sc_kernel():
    mesh = plsc.VectorSubcoreMesh(core_axis_name="c", subcore_axis_name="s")

    @functools.partial(
        pl.kernel,
        out_type=jax.ShapeDtypeStruct((_N,), jnp.float32),
        mesh=mesh,
        scratch_types=[
            pltpu.VMEM((_K,), jnp.float32),
            pltpu.VMEM((_K,), jnp.float32),
            pltpu.VMEM((_K,), jnp.int32),
            pltpu.VMEM((_K,), jnp.int32),
            pltpu.VMEM((_K,), jnp.float32),
            pltpu.VMEM((_K,), jnp.float32),
            pltpu.SemaphoreType.DMA((2,)),
            pltpu.SemaphoreType.DMA((2,)),
            pltpu.SemaphoreType.DMA((2,)),
        ],
    )
    def _sc_body(d_hbm, x_hbm, out_hbm, xv0, xv1, idx0, idx1, val0, val1,
                 sem_x, sem_g, sem_o):
        xv = [xv0, xv1]
        idx = [idx0, idx1]
        val = [val0, val1]
        wid = lax.axis_index("s") * _NC + lax.axis_index("c")
        base = wid * _B_PER_W

        cp_x = [None, None]
        cp_g = [None, None]
        cp_o = [None, None]
        cp_x[0] = pltpu.async_copy(
            x_hbm.at[pl.ds(base, _K)], xv[0], sem_x.at[0])
        for k in range(_NCH):
            b = k % 2
            if k + 1 < _NCH:
                cp_x[1 - b] = pltpu.async_copy(
                    x_hbm.at[pl.ds(base + (k + 1) * _K, _K)],
                    xv[1 - b], sem_x.at[1 - b])
            cp_x[b].wait()

            xv_b = xv[b]
            idx_b = idx[b]

            @plsc.parallel_loop(0, _K, _L, unroll=8)
            def _compute_idx(i):
                p = xv_b[pl.ds(i, _L)] * _SCALE + 1.0
                p = jnp.maximum(p, 1.0)
                p = jnp.minimum(p, _MAX_IND)
                idx_b[pl.ds(i, _L)] = p.astype(jnp.int32)

            if k >= 1:
                cp_g[1 - b].wait()
                cp_o[1 - b] = pltpu.async_copy(
                    val[1 - b], out_hbm.at[pl.ds(base + (k - 1) * _K, _K)],
                    sem_o.at[1 - b])
            if k >= 2:
                cp_o[b].wait()
            cp_g[b] = pltpu.async_copy(
                d_hbm.at[idx[b]], val[b], sem_g.at[b])

        bl = (_NCH - 1) % 2
        cp_g[bl].wait()
        cp_o[bl] = pltpu.async_copy(
            val[bl], out_hbm.at[pl.ds(base + (_NCH - 1) * _K, _K)],
            sem_o.at[bl])
        if _NCH >= 2:
            cp_o[1 - bl].wait()
        cp_o[bl].wait()

    return _sc_body


def kernel(input_data, weight_vec):
    w2 = weight_vec.reshape(_R, _C)
    d2 = pl.pallas_call(
        _tc_precompute,
        grid=(_G,),
        in_specs=[
            pl.BlockSpec((_RB, _C), lambda i: (i, 0)),
            pl.BlockSpec(
                (8, _C),
                lambda i: (jnp.minimum((i + 1) * (_RB // 8), _R // 8 - 1), 0)),
        ],
        out_specs=pl.BlockSpec((_RB, _C), lambda i: (i, 0)),
        out_shape=jax.ShapeDtypeStruct((_R, _C), jnp.float32),
    )(w2, w2)
    return _make_sc_kernel()(d2.reshape(_N), input_data)


# speedup vs baseline: 9837.5373x; 1.1360x over previous
import functools

import jax
import jax.numpy as jnp
from jax import lax
from jax.experimental import pallas as pl
from jax.experimental.pallas import tpu as pltpu
from jax.experimental.pallas import tpu_sc as plsc

_GEN = 64
_N = 1048576
_R = 8192
_C = 128
_RB = 2048
_G = _R // _RB
_NUM_ASSOC = _N + 1 - _GEN
_SCALE = float(_NUM_ASSOC - 2)
_MAX_IND = float(_NUM_ASSOC - 1)

_NC = 2
_NS = 16
_NW = _NC * _NS
_B_PER_W = _N // _NW
_K = 8192
_NCH = _B_PER_W // _K
_L = 16


def _tc_precompute(w_ref, wn_ref, d_ref):
    w = w_ref[...]
    wsh = jnp.concatenate([w[1:, :], wn_ref[0:1, :]], axis=0)
    ii = lax.broadcasted_iota(jnp.int32, (_C, _C), 0)
    cc = lax.broadcasted_iota(jnp.int32, (_C, _C), 1)
    a_mask = ((ii >= cc) & (ii <= cc + 63)).astype(jnp.float32)
    b_mask = (ii <= cc - 65).astype(jnp.float32)
    d_ref[...] = (
        jnp.dot(w, a_mask, preferred_element_type=jnp.float32)
        + jnp.dot(wsh, b_mask, preferred_element_type=jnp.float32))


@functools.cache
def _make_sc_kernel():
    mesh = plsc.VectorSubcoreMesh(core_axis_name="c", subcore_axis_name="s")

    @functools.partial(
        pl.kernel,
        out_type=jax.ShapeDtypeStruct((_N,), jnp.float32),
        mesh=mesh,
        scratch_types=[
            pltpu.VMEM((_K,), jnp.float32),
            pltpu.VMEM((_K,), jnp.float32),
            pltpu.VMEM((_K,), jnp.int32),
            pltpu.VMEM((_K,), jnp.int32),
            pltpu.VMEM((_K,), jnp.float32),
            pltpu.VMEM((_K,), jnp.float32),
            pltpu.SemaphoreType.DMA((2,)),
            pltpu.SemaphoreType.DMA((2,)),
            pltpu.SemaphoreType.DMA((2,)),
        ],
    )
    def _sc_body(d_hbm, x_hbm, out_hbm, xv0, xv1, idx0, idx1, val0, val1,
                 sem_x, sem_g, sem_o):
        xv = [xv0, xv1]
        idx = [idx0, idx1]
        val = [val0, val1]
        wid = lax.axis_index("s") * _NC + lax.axis_index("c")
        base = wid * _B_PER_W

        cp_x = [None, None]
        cp_g = [None, None]
        cp_o = [None, None]
        cp_x[0] = pltpu.async_copy(
            x_hbm.at[pl.ds(base, _K)], xv[0], sem_x.at[0])
        for k in range(_NCH):
            b = k % 2
            if k + 1 < _NCH:
                cp_x[1 - b] = pltpu.async_copy(
                    x_hbm.at[pl.ds(base + (k + 1) * _K, _K)],
                    xv[1 - b], sem_x.at[1 - b])
            cp_x[b].wait()

            xv_b = xv[b]
            idx_b = idx[b]

            @plsc.parallel_loop(0, _K, _L, unroll=8)
            def _compute_idx(i):
                p = xv_b[pl.ds(i, _L)] * _SCALE + 1.0
                p = jnp.maximum(p, 1.0)
                p = jnp.minimum(p, _MAX_IND)
                idx_b[pl.ds(i, _L)] = p.astype(jnp.int32)

            if k >= 1:
                cp_g[1 - b].wait()
                cp_o[1 - b] = pltpu.async_copy(
                    val[1 - b], out_hbm.at[pl.ds(base + (k - 1) * _K, _K)],
                    sem_o.at[1 - b])
            if k >= 2:
                cp_o[b].wait()
            cp_g[b] = pltpu.async_copy(
                d_hbm.at[idx[b]], val[b], sem_g.at[b])

        bl = (_NCH - 1) % 2
        cp_g[bl].wait()
        cp_o[bl] = pltpu.async_copy(
            val[bl], out_hbm.at[pl.ds(base + (_NCH - 1) * _K, _K)],
            sem_o.at[bl])
        if _NCH >= 2:
            cp_o[1 - bl].wait()
        cp_o[bl].wait()

    return _sc_body


def kernel(input_data, weight_vec):
    w2 = weight_vec.reshape(_R, _C)
    d2 = pl.pallas_call(
        _tc_precompute,
        grid=(_G,),
        in_specs=[
            pl.BlockSpec((_RB, _C), lambda i: (i, 0)),
            pl.BlockSpec(
                (8, _C),
                lambda i: (jnp.minimum((i + 1) * (_RB // 8), _R // 8 - 1), 0)),
        ],
        out_specs=pl.BlockSpec((_RB, _C), lambda i: (i, 0)),
        out_shape=jax.ShapeDtypeStruct((_R, _C), jnp.float32),
    )(w2, w2)
    return _make_sc_kernel()(d2.reshape(_N), input_data)


# speedup vs baseline: 10043.1432x; 1.0209x over previous
import functools

import jax
import jax.numpy as jnp
from jax import lax
from jax.experimental import pallas as pl
from jax.experimental.pallas import tpu as pltpu
from jax.experimental.pallas import tpu_sc as plsc

_GEN = 64
_N = 1048576
_R = 8192
_C = 128
_RB = 2048
_G = _R // _RB
_NUM_ASSOC = _N + 1 - _GEN
_SCALE = float(_NUM_ASSOC - 2)
_MAX_IND = float(_NUM_ASSOC - 1)

_NC = 2
_NS = 16
_NW = _NC * _NS
_B_PER_W = _N // _NW
_K = 4096
_NCH = _B_PER_W // _K
_NB = 3
_L = 16


def _tc_precompute(w_ref, wn_ref, d_ref):
    w = w_ref[...]
    wsh = jnp.concatenate([w[1:, :], wn_ref[0:1, :]], axis=0)
    ii = lax.broadcasted_iota(jnp.int32, (_C, _C), 0)
    cc = lax.broadcasted_iota(jnp.int32, (_C, _C), 1)
    a_mask = ((ii >= cc) & (ii <= cc + 63)).astype(jnp.float32)
    b_mask = (ii <= cc - 65).astype(jnp.float32)
    d_ref[...] = (
        jnp.dot(w, a_mask, preferred_element_type=jnp.float32)
        + jnp.dot(wsh, b_mask, preferred_element_type=jnp.float32))


@functools.cache
def _make_sc_kernel():
    mesh = plsc.VectorSubcoreMesh(core_axis_name="c", subcore_axis_name="s")

    @functools.partial(
        pl.kernel,
        out_type=jax.ShapeDtypeStruct((_N,), jnp.float32),
        mesh=mesh,
        scratch_types=(
            [pltpu.VMEM((_K,), jnp.float32) for _ in range(2)]
            + [pltpu.VMEM((_K,), jnp.int32) for _ in range(_NB)]
            + [pltpu.VMEM((_K,), jnp.float32) for _ in range(_NB)]
            + [pltpu.SemaphoreType.DMA((2,)),
               pltpu.SemaphoreType.DMA((_NB,)),
               pltpu.SemaphoreType.DMA((_NB,))]
        ),
    )
    def _sc_body(d_hbm, x_hbm, out_hbm, *refs):
        xv = list(refs[0:2])
        idx = list(refs[2:2 + _NB])
        val = list(refs[2 + _NB:2 + 2 * _NB])
        sem_x, sem_g, sem_o = refs[2 + 2 * _NB:]
        wid = lax.axis_index("s") * _NC + lax.axis_index("c")
        base = wid * _B_PER_W

        cp_x = [None, None]
        cp_g = [None] * _NCH
        cp_o = [None] * _NCH
        cp_x[0] = pltpu.async_copy(
            x_hbm.at[pl.ds(base, _K)], xv[0], sem_x.at[0])
        for k in range(_NCH):
            b2 = k % 2
            b = k % _NB
            if k + 1 < _NCH:
                cp_x[1 - b2] = pltpu.async_copy(
                    x_hbm.at[pl.ds(base + (k + 1) * _K, _K)],
                    xv[1 - b2], sem_x.at[1 - b2])
            cp_x[b2].wait()

            xv_b = xv[b2]
            idx_b = idx[b]

            @plsc.parallel_loop(0, _K, _L, unroll=8)
            def _compute_idx(i):
                p = xv_b[pl.ds(i, _L)] * _SCALE + 1.0
                p = jnp.maximum(p, 1.0)
                p = jnp.minimum(p, _MAX_IND)
                idx_b[pl.ds(i, _L)] = p.astype(jnp.int32)

            if k >= 2:
                cp_g[k - 2].wait()
                cp_o[k - 2] = pltpu.async_copy(
                    val[(k - 2) % _NB],
                    out_hbm.at[pl.ds(base + (k - 2) * _K, _K)],
                    sem_o.at[(k - 2) % _NB])
            if k >= _NB:
                cp_o[k - _NB].wait()
            cp_g[k] = pltpu.async_copy(
                d_hbm.at[idx_b], val[b], sem_g.at[b])

        for k in range(max(_NCH - 2, 0), _NCH):
            cp_g[k].wait()
            cp_o[k] = pltpu.async_copy(
                val[k % _NB], out_hbm.at[pl.ds(base + k * _K, _K)],
                sem_o.at[k % _NB])
        for k in range(max(_NCH - _NB, 0), _NCH):
            cp_o[k].wait()

    return _sc_body


def kernel(input_data, weight_vec):
    w2 = weight_vec.reshape(_R, _C)
    d2 = pl.pallas_call(
        _tc_precompute,
        grid=(_G,),
        in_specs=[
            pl.BlockSpec((_RB, _C), lambda i: (i, 0)),
            pl.BlockSpec(
                (8, _C),
                lambda i: (jnp.minimum((i + 1) * (_RB // 8), _R // 8 - 1), 0)),
        ],
        out_specs=pl.BlockSpec((_RB, _C), lambda i: (i, 0)),
        out_shape=jax.ShapeDtypeStruct((_R, _C), jnp.float32),
    )(w2, w2)
    return _make_sc_kernel()(d2.reshape(_N), input_data)


# speedup vs baseline: 16839.8974x; 1.6768x over previous
import functools

import jax
import jax.numpy as jnp
from jax import lax
from jax.experimental import pallas as pl
from jax.experimental.pallas import tpu as pltpu
from jax.experimental.pallas import tpu_sc as plsc

_GEN = 64
_N = 1048576
_R = 8192
_C = 128
_RB = 2048
_G = _R // _RB
_NUM_ASSOC = _N + 1 - _GEN
_SCALE = float(_NUM_ASSOC - 2)
_MAX_IND = float(_NUM_ASSOC - 1)

_NC = 2
_NS = 16
_NW = _NC * _NS
_B_PER_W = _N // _NW
_K = 4096
_NCH = _B_PER_W // _K
_NB = 3
_L = 16


def _tc_precompute(w_ref, wn_ref, d_ref):
    w = w_ref[...]
    wsh = jnp.concatenate([w[1:, :], wn_ref[0:1, :]], axis=0)
    ii = lax.broadcasted_iota(jnp.int32, (_C, _C), 0)
    cc = lax.broadcasted_iota(jnp.int32, (_C, _C), 1)
    a_mask = ((ii >= cc) & (ii <= cc + 63)).astype(jnp.float32)
    b_mask = (ii <= cc - 65).astype(jnp.float32)
    d_ref[...] = (
        jnp.dot(w, a_mask, preferred_element_type=jnp.float32)
        + jnp.dot(wsh, b_mask, preferred_element_type=jnp.float32))


@functools.cache
def _make_sc_kernel():
    mesh = plsc.VectorSubcoreMesh(core_axis_name="c", subcore_axis_name="s")

    @functools.partial(
        pl.kernel,
        out_type=jax.ShapeDtypeStruct((_N,), jnp.float32),
        mesh=mesh,
        scratch_types=(
            [pltpu.VMEM((_K,), jnp.float32) for _ in range(2)]
            + [pltpu.VMEM((_K,), jnp.int32) for _ in range(_NB)]
            + [pltpu.VMEM((_K,), jnp.float32) for _ in range(_NB)]
            + [pltpu.SemaphoreType.DMA((2,)),
               pltpu.SemaphoreType.DMA((_NB,)),
               pltpu.SemaphoreType.DMA((_NB,)),
               pltpu.VMEM_SHARED((_N,), jnp.float32)]
        ),
    )
    def _sc_body(d_hbm, x_hbm, out_hbm, *refs):
        xv = list(refs[0:2])
        idx = list(refs[2:2 + _NB])
        val = list(refs[2 + _NB:2 + 2 * _NB])
        sem_x, sem_g, sem_o, dsh = refs[2 + 2 * _NB:]
        sid = lax.axis_index("s")
        wid = sid * _NC + lax.axis_index("c")
        base = wid * _B_PER_W

        cp_x = [None, None]
        cp_g = [None] * _NCH
        cp_o = [None] * _NCH
        cp_x[0] = pltpu.async_copy(
            x_hbm.at[pl.ds(base, _K)], xv[0], sem_x.at[0])

        dslice = _N // _NS
        pltpu.sync_copy(d_hbm.at[pl.ds(sid * dslice, dslice)],
                        dsh.at[pl.ds(sid * dslice, dslice)])
        plsc.subcore_barrier()
        for k in range(_NCH):
            b2 = k % 2
            b = k % _NB
            if k + 1 < _NCH:
                cp_x[1 - b2] = pltpu.async_copy(
                    x_hbm.at[pl.ds(base + (k + 1) * _K, _K)],
                    xv[1 - b2], sem_x.at[1 - b2])
            cp_x[b2].wait()

            xv_b = xv[b2]
            idx_b = idx[b]

            @plsc.parallel_loop(0, _K, _L, unroll=8)
            def _compute_idx(i):
                p = xv_b[pl.ds(i, _L)] * _SCALE + 1.0
                p = jnp.maximum(p, 1.0)
                p = jnp.minimum(p, _MAX_IND)
                idx_b[pl.ds(i, _L)] = p.astype(jnp.int32)

            if k >= 2:
                cp_g[k - 2].wait()
                cp_o[k - 2] = pltpu.async_copy(
                    val[(k - 2) % _NB],
                    out_hbm.at[pl.ds(base + (k - 2) * _K, _K)],
                    sem_o.at[(k - 2) % _NB])
            if k >= _NB:
                cp_o[k - _NB].wait()
            cp_g[k] = pltpu.async_copy(
                dsh.at[idx_b], val[b], sem_g.at[b])

        for k in range(max(_NCH - 2, 0), _NCH):
            cp_g[k].wait()
            cp_o[k] = pltpu.async_copy(
                val[k % _NB], out_hbm.at[pl.ds(base + k * _K, _K)],
                sem_o.at[k % _NB])
        for k in range(max(_NCH - _NB, 0), _NCH):
            cp_o[k].wait()

    return _sc_body


def kernel(input_data, weight_vec):
    w2 = weight_vec.reshape(_R, _C)
    d2 = pl.pallas_call(
        _tc_precompute,
        grid=(_G,),
        in_specs=[
            pl.BlockSpec((_RB, _C), lambda i: (i, 0)),
            pl.BlockSpec(
                (8, _C),
                lambda i: (jnp.minimum((i + 1) * (_RB // 8), _R // 8 - 1), 0)),
        ],
        out_specs=pl.BlockSpec((_RB, _C), lambda i: (i, 0)),
        out_shape=jax.ShapeDtypeStruct((_R, _C), jnp.float32),
    )(w2, w2)
    return _make_sc_kernel()(d2.reshape(_N), input_data)
